# baseline (speedup 1.0000x reference)
import functools

import jax
import jax.numpy as jnp
from jax import lax
from jax.experimental import pallas as pl
from jax.experimental.pallas import tpu as pltpu
from jax.experimental.pallas import tpu_sc as plsc

N = 10000
E = 320000
F = 128
H = 256
O = 256
C = 16
G = 64

NPAD = 10240
EPAD = 327680
NC = 2
NT = 16
LN = 16
CHUNK = 64
AGG_CHUNKS = EPAD // (NT * CHUNK)
GC = 16
NGRP = AGG_CHUNKS // GC
DEG_CHUNKS = EPAD // (NC * NT * CHUNK)
ROWS_PER_TILE = NPAD // NT
QF = F // 2
RB = 1024
NBLK = NPAD // RB

_mesh = plsc.VectorSubcoreMesh(
    core_axis_name="c", subcore_axis_name="s", num_cores=NC, num_subcores=NT)


def _deg_body(dst_hbm, out_hbm, idx_v, ones_v, zero_v, acc_sh, sem):
  c = lax.axis_index("c")
  s = lax.axis_index("s")
  w = c * NT + s
  for k in range(CHUNK // LN):
    ones_v[pl.ds(k * LN, LN)] = jnp.ones((LN,), jnp.float32)
  for k in range(ROWS_PER_TILE // LN):
    zero_v[pl.ds(k * LN, LN)] = jnp.zeros((LN,), jnp.float32)
  pltpu.sync_copy(zero_v, acc_sh.at[pl.ds(s * ROWS_PER_TILE, ROWS_PER_TILE)])
  pltpu.async_copy(dst_hbm.at[w], idx_v, sem).wait()
  plsc.subcore_barrier()

  def chunk(j, carry):
    pltpu.sync_copy(ones_v, acc_sh.at[idx_v.at[j]], add=True)
    return carry

  lax.fori_loop(0, DEG_CHUNKS, chunk, 0)
  plsc.subcore_barrier()
  sl = pl.ds(s * ROWS_PER_TILE, ROWS_PER_TILE)
  pltpu.sync_copy(acc_sh.at[sl], out_hbm.at[c].at[sl])


_deg_kernel = pl.kernel(
    _deg_body,
    out_type=jax.ShapeDtypeStruct((NC, NPAD), jnp.float32),
    mesh=_mesh,
    scratch_types=[
        pltpu.VMEM((DEG_CHUNKS, CHUNK), jnp.int32),
        pltpu.VMEM((CHUNK,), jnp.float32),
        pltpu.VMEM((ROWS_PER_TILE,), jnp.float32),
        pltpu.VMEM_SHARED((NPAD,), jnp.float32),
        pltpu.SemaphoreType.DMA,
    ],
)


def _agg_body(u_hbm, src_hbm, dst_hbm, out_hbm, src_g, dst_g, gbuf,
              acc_sh, gsem, isem, ssem):
  del out_hbm
  c = lax.axis_index("c")
  s = lax.axis_index("s")
  src_t = src_hbm.at[s]
  dst_t = dst_hbm.at[s]
  sl = pl.ds(s * ROWS_PER_TILE, ROWS_PER_TILE)

  def istart(g, p):
    gsl = pl.ds(g * GC, GC)
    pltpu.make_async_copy(src_t.at[gsl], src_g.at[p], isem).start()
    pltpu.make_async_copy(dst_t.at[gsl], dst_g.at[p], isem).start()

  def iwait(g, p):
    gsl = pl.ds(g * GC, GC)
    pltpu.make_async_copy(src_t.at[gsl], src_g.at[p], isem).wait()
    pltpu.make_async_copy(dst_t.at[gsl], dst_g.at[p], isem).wait()

  def _pk(j):
    return lax.rem(lax.div(j, GC), 2), lax.rem(j, GC)

  def gdesc(j, b, uq):
    p, k = _pk(j)
    return pltpu.make_async_copy(uq.at[src_g.at[p, k]], gbuf.at[b], gsem)

  def sdesc(j, b):
    p, k = _pk(j)
    return pltpu.make_async_copy(gbuf.at[b], acc_sh.at[dst_g.at[p, k]], ssem)

  for q in range(1):
    uq = u_hbm
    istart(0, 0)
    iwait(0, 0)
    plsc.subcore_barrier()
    gdesc(jnp.int32(0), jnp.int32(0), uq).start()

    def body(j, carry):
      b = lax.rem(j, 2)
      nb = 1 - b
      g = lax.div(j, GC)
      gdesc(j, b, uq).wait()


      @pl.when(jnp.logical_and(lax.rem(j, GC) == 0, g + 1 < NGRP))
      def _():
        istart(g + 1, lax.rem(g + 1, 2))


      @pl.when(j + 1 < AGG_CHUNKS)
      def _():
        g1 = lax.div(j + 1, GC)

        @pl.when(lax.rem(j + 1, GC) == 0)
        def _():
          iwait(g1, lax.rem(g1, 2))

        gdesc(j + 1, nb, uq).start()

      return carry

    lax.fori_loop(0, AGG_CHUNKS, body, 0)
    plsc.subcore_barrier()


_agg_kernel = pl.kernel(
    _agg_body,
    out_type=jax.ShapeDtypeStruct((2 * NC, NPAD, QF), jnp.float32),
    mesh=_mesh,
    scratch_types=[
        pltpu.VMEM((2, GC, CHUNK), jnp.int32),
        pltpu.VMEM((2, GC, CHUNK), jnp.int32),
        pltpu.VMEM((2, CHUNK, 2 * F), jnp.float32),
        pltpu.VMEM_SHARED((NPAD, QF), jnp.float32),
        pltpu.SemaphoreType.DMA,
        pltpu.SemaphoreType.DMA,
        pltpu.SemaphoreType.DMA,
    ],
)


def _mm1_body(deg_ref, x_ref, w_ref, u_ref, dis_ref):
  d = deg_ref[0] + deg_ref[1] + 1.0
  dis = lax.rsqrt(d)
  dis_ref[...] = dis
  xw = jnp.dot(x_ref[...], w_ref[...], preferred_element_type=jnp.float32)
  u = dis * xw
  for q in range(4):
    u_ref[q] = u[:, q * QF:(q + 1) * QF]


def _mm1(deg3, x, W1):
  return pl.pallas_call(
      _mm1_body,
      grid=(NBLK,),
      in_specs=[
          pl.BlockSpec((NC, RB, 1), lambda i: (0, i, 0)),
          pl.BlockSpec((RB, F), lambda i: (i, 0)),
          pl.BlockSpec((F, H), lambda i: (0, 0)),
      ],
      out_specs=[
          pl.BlockSpec((4, RB, QF), lambda i: (0, i, 0)),
          pl.BlockSpec((RB, 1), lambda i: (i, 0)),
      ],
      out_shape=[
          jax.ShapeDtypeStruct((4, NPAD, QF), jnp.float32),
          jax.ShapeDtypeStruct((NPAD, 1), jnp.float32),
      ],
  )(deg3, x, W1)


def _mm2_body(a_ref, dis_ref, b_ref, w_ref, u_ref):
  h = jnp.concatenate([a_ref[q] for q in range(4)], axis=1)
  dis = dis_ref[...]
  h1 = jnp.maximum(dis * h + b_ref[...], 0.0)
  hw = jnp.dot(h1, w_ref[...], preferred_element_type=jnp.float32)
  u = dis * hw
  for q in range(4):
    u_ref[q] = u[:, q * QF:(q + 1) * QF]


def _mm2(a1, dis, b1r, W2):
  return pl.pallas_call(
      _mm2_body,
      grid=(NBLK,),
      in_specs=[
          pl.BlockSpec((4, RB, QF), lambda i: (0, i, 0)),
          pl.BlockSpec((RB, 1), lambda i: (i, 0)),
          pl.BlockSpec((1, H), lambda i: (0, 0)),
          pl.BlockSpec((H, O), lambda i: (0, 0)),
      ],
      out_specs=pl.BlockSpec((4, RB, QF), lambda i: (0, i, 0)),
      out_shape=jax.ShapeDtypeStruct((4, NPAD, QF), jnp.float32),
  )(a1, dis, b1r, W2)


def _pool_body(a_ref, dis_ref, b_ref, batch_ref, fcw_ref, fcb_ref, out_ref,
               pooled):
  i = pl.program_id(0)

  @pl.when(i == 0)
  def _():
    pooled[...] = jnp.full((G, O), -jnp.inf, jnp.float32)

  h = jnp.concatenate([a_ref[q] for q in range(4)], axis=1)
  h2 = jnp.maximum(dis_ref[...] * h + b_ref[...], 0.0)
  bcol = batch_ref[...]

  def seg(g, carry):
    m = jnp.where(bcol == g, h2, -jnp.inf)
    mx = jnp.max(m, axis=0, keepdims=True)
    pooled[pl.ds(g, 1), :] = jnp.maximum(pooled[pl.ds(g, 1), :], mx)
    return carry

  lax.fori_loop(0, G, seg, 0)

  @pl.when(i == NBLK - 1)
  def _():
    out_ref[...] = jnp.dot(pooled[...], fcw_ref[...],
                           preferred_element_type=jnp.float32) + fcb_ref[...]


def _pool(a2, dis, b2r, batchc, fcWp, fcbp):
  return pl.pallas_call(
      _pool_body,
      grid=(NBLK,),
      in_specs=[
          pl.BlockSpec((4, RB, QF), lambda i: (0, i, 0)),
          pl.BlockSpec((RB, 1), lambda i: (i, 0)),
          pl.BlockSpec((1, O), lambda i: (0, 0)),
          pl.BlockSpec((RB, 1), lambda i: (i, 0)),
          pl.BlockSpec((O, 128), lambda i: (0, 0)),
          pl.BlockSpec((1, 128), lambda i: (0, 0)),
      ],
      out_specs=pl.BlockSpec((G, 128), lambda i: (0, 0)),
      out_shape=jax.ShapeDtypeStruct((G, 128), jnp.float32),
      scratch_shapes=[pltpu.VMEM((G, O), jnp.float32)],
  )(a2, dis, b2r, batchc, fcWp, fcbp)


@jax.jit
def kernel(x, edge_index, batch, W1, b1, W2, b2, fcW, fcb):
  pad_e = EPAD - E
  src = jnp.concatenate(
      [edge_index[0], jnp.full((pad_e,), N, jnp.int32)]).reshape(NT, AGG_CHUNKS,
                                                                 CHUNK)
  dst = jnp.concatenate(
      [edge_index[1], jnp.full((pad_e,), N, jnp.int32)]).reshape(NT, AGG_CHUNKS,
                                                                 CHUNK)
  dst_deg = dst.reshape(NC * NT, DEG_CHUNKS, CHUNK)
  xp = jnp.pad(x, ((0, NPAD - N), (0, 0)))
  batchc = jnp.pad(batch, (0, NPAD - N), constant_values=G)[:, None]
  b1r = b1[None, :]
  b2r = b2[None, :]
  fcWp = jnp.pad(fcW, ((0, 0), (0, 128 - C)))
  fcbp = jnp.pad(fcb, (0, 128 - C))[None, :]

  deg = _deg_kernel(dst_deg)
  deg3 = deg[:, :, None]
  u1, dis = _mm1(deg3, xp, W1)
  a1 = _agg_kernel(jnp.concatenate([u1[0], u1[1], u1[2], u1[3]], axis=1), src, dst)
  u2 = _mm2(a1, dis, b1r, W2)
  a2 = _agg_kernel(jnp.concatenate([u2[0], u2[1], u2[2], u2[3]], axis=1), src, dst)
  outp = _pool(a2, dis, b2r, batchc, fcWp, fcbp)
  return outp[:, :C]


# speedup vs baseline: 1.5332x; 1.5332x over previous
import jax
import jax.numpy as jnp
from jax import lax
from jax.experimental import pallas as pl
from jax.experimental.pallas import tpu as pltpu
from jax.experimental.pallas import tpu_sc as plsc

N = 10000
E = 320000
F = 128
H = 256
O = 256
C = 16
G = 64

NPAD = 10240
EPAD = 327680
NC = 2
NT = 16
LN = 16
CHUNK = 64
NB = 4
AGG_CHUNKS = EPAD // (NT * CHUNK)
GC = 16
NGRP = AGG_CHUNKS // GC
DEG_CHUNKS = EPAD // (NC * NT * CHUNK)
ROWS_PER_TILE = NPAD // NT
RB = 1024
NBLK = NPAD // RB

_mesh = plsc.VectorSubcoreMesh(
    core_axis_name="c", subcore_axis_name="s", num_cores=NC, num_subcores=NT)


def _deg_body(dst_hbm, out_hbm, idx_v, ones_v, zero_v, acc_sh, sem):
  c = lax.axis_index("c")
  s = lax.axis_index("s")
  w = c * NT + s
  for k in range(CHUNK // LN):
    ones_v[pl.ds(k * LN, LN)] = jnp.ones((LN,), jnp.float32)
  for k in range(ROWS_PER_TILE // LN):
    zero_v[pl.ds(k * LN, LN)] = jnp.zeros((LN,), jnp.float32)
  pltpu.sync_copy(zero_v, acc_sh.at[pl.ds(s * ROWS_PER_TILE, ROWS_PER_TILE)])
  pltpu.async_copy(dst_hbm.at[w], idx_v, sem).wait()
  plsc.subcore_barrier()

  def chunk(j, carry):
    pltpu.sync_copy(ones_v, acc_sh.at[idx_v.at[j]], add=True)
    return carry

  lax.fori_loop(0, DEG_CHUNKS, chunk, 0)
  plsc.subcore_barrier()
  sl = pl.ds(s * ROWS_PER_TILE, ROWS_PER_TILE)
  pltpu.sync_copy(acc_sh.at[sl], out_hbm.at[c].at[sl])


_deg_kernel = pl.kernel(
    _deg_body,
    out_type=jax.ShapeDtypeStruct((NC, NPAD), jnp.float32),
    mesh=_mesh,
    scratch_types=[
        pltpu.VMEM((DEG_CHUNKS, CHUNK), jnp.int32),
        pltpu.VMEM((CHUNK,), jnp.float32),
        pltpu.VMEM((ROWS_PER_TILE,), jnp.float32),
        pltpu.VMEM_SHARED((NPAD,), jnp.float32),
        pltpu.SemaphoreType.DMA,
    ],
)


def _agg_body(u_hbm, src_hbm, dst_hbm, out_hbm, src_g, dst_g, gbuf, acc_sh,
              gsem, isem, ssem):
  c = lax.axis_index("c")
  s = lax.axis_index("s")
  u_c = u_hbm.at[c]
  src_t = src_hbm.at[s]
  dst_t = dst_hbm.at[s]
  sl = pl.ds(s * ROWS_PER_TILE, ROWS_PER_TILE)
  pltpu.sync_copy(u_c.at[sl], acc_sh.at[sl])

  def istart(g, p):
    gsl = pl.ds(g * GC, GC)
    pltpu.make_async_copy(src_t.at[gsl], src_g.at[p], isem).start()
    pltpu.make_async_copy(dst_t.at[gsl], dst_g.at[p], isem).start()

  def iwait(g, p):
    gsl = pl.ds(g * GC, GC)
    pltpu.make_async_copy(src_t.at[gsl], src_g.at[p], isem).wait()
    pltpu.make_async_copy(dst_t.at[gsl], dst_g.at[p], isem).wait()

  def _pk(j):
    return lax.rem(lax.div(j, GC), 2), lax.rem(j, GC)

  def gdesc(j):
    p, k = _pk(j)
    b = lax.rem(j, NB)
    return pltpu.make_async_copy(u_c.at[src_g.at[p, k]], gbuf.at[b], gsem)

  def sdesc(j):
    p, k = _pk(j)
    b = lax.rem(j, NB)
    return pltpu.make_async_copy(gbuf.at[b], acc_sh.at[dst_g.at[p, k]], ssem)

  istart(0, 0)
  iwait(0, 0)
  plsc.subcore_barrier()
  gdesc(jnp.int32(0)).start()
  gdesc(jnp.int32(1)).start()
  gdesc(jnp.int32(2)).start()

  def body(j, carry):
    g = lax.div(j, GC)
    gdesc(j).wait()

    @pl.when(j >= 1)
    def _():
      sdesc(j - 1).wait()

    sdesc(j).start(add=True)

    @pl.when(jnp.logical_and(lax.rem(j, GC) == 0, g + 1 < NGRP))
    def _():
      istart(g + 1, lax.rem(g + 1, 2))

    @pl.when(j + NB - 1 < AGG_CHUNKS)
    def _():
      g3 = lax.div(j + NB - 1, GC)

      @pl.when(lax.rem(j + NB - 1, GC) == 0)
      def _():
        iwait(g3, lax.rem(g3, 2))

      gdesc(j + NB - 1).start()

    return carry

  lax.fori_loop(0, AGG_CHUNKS, body, 0)
  sdesc(jnp.int32(AGG_CHUNKS - 1)).wait()
  plsc.subcore_barrier()
  pltpu.sync_copy(acc_sh.at[sl], out_hbm.at[c].at[sl])


_agg_kernel = pl.kernel(
    _agg_body,
    out_type=jax.ShapeDtypeStruct((NC, NPAD, F), jnp.float32),
    mesh=_mesh,
    scratch_types=[
        pltpu.VMEM((2, GC, CHUNK), jnp.int32),
        pltpu.VMEM((2, GC, CHUNK), jnp.int32),
        pltpu.VMEM((NB, CHUNK, F), jnp.float32),
        pltpu.VMEM_SHARED((NPAD, F), jnp.float32),
        pltpu.SemaphoreType.DMA,
        pltpu.SemaphoreType.DMA,
        pltpu.SemaphoreType.DMA,
    ],
)


def _mm1_body(deg_ref, x_ref, w_ref, u_ref, dis_ref):
  d = deg_ref[0] + deg_ref[1] + 1.0
  dis = lax.rsqrt(d)
  dis_ref[...] = dis
  xw = jnp.dot(x_ref[...], w_ref[...], preferred_element_type=jnp.float32)
  u = dis * xw
  u_ref[0] = u[:, :F]
  u_ref[1] = u[:, F:]


def _mm1(deg3, x, W1):
  return pl.pallas_call(
      _mm1_body,
      grid=(NBLK,),
      in_specs=[
          pl.BlockSpec((NC, RB, 1), lambda i: (0, i, 0)),
          pl.BlockSpec((RB, F), lambda i: (i, 0)),
          pl.BlockSpec((F, H), lambda i: (0, 0)),
      ],
      out_specs=[
          pl.BlockSpec((NC, RB, F), lambda i: (0, i, 0)),
          pl.BlockSpec((RB, 1), lambda i: (i, 0)),
      ],
      out_shape=[
          jax.ShapeDtypeStruct((NC, NPAD, F), jnp.float32),
          jax.ShapeDtypeStruct((NPAD, 1), jnp.float32),
      ],
  )(deg3, x, W1)


def _mm2_body(a_ref, dis_ref, b_ref, w_ref, u_ref):
  h = jnp.concatenate([a_ref[0], a_ref[1]], axis=1)
  dis = dis_ref[...]
  h1 = jnp.maximum(dis * h + b_ref[...], 0.0)
  hw = jnp.dot(h1, w_ref[...], preferred_element_type=jnp.float32)
  u = dis * hw
  u_ref[0] = u[:, :F]
  u_ref[1] = u[:, F:]


def _mm2(a1, dis, b1r, W2):
  return pl.pallas_call(
      _mm2_body,
      grid=(NBLK,),
      in_specs=[
          pl.BlockSpec((NC, RB, F), lambda i: (0, i, 0)),
          pl.BlockSpec((RB, 1), lambda i: (i, 0)),
          pl.BlockSpec((1, H), lambda i: (0, 0)),
          pl.BlockSpec((H, O), lambda i: (0, 0)),
      ],
      out_specs=pl.BlockSpec((NC, RB, F), lambda i: (0, i, 0)),
      out_shape=jax.ShapeDtypeStruct((NC, NPAD, F), jnp.float32),
  )(a1, dis, b1r, W2)


def _pool_body(a_ref, dis_ref, b_ref, batch_ref, fcw_ref, fcb_ref, out_ref,
               pooled):
  i = pl.program_id(0)

  @pl.when(i == 0)
  def _():
    pooled[...] = jnp.full((G, O), -jnp.inf, jnp.float32)

  h = jnp.concatenate([a_ref[0], a_ref[1]], axis=1)
  h2 = jnp.maximum(dis_ref[...] * h + b_ref[...], 0.0)
  bcol = batch_ref[...]

  def seg(g, carry):
    m = jnp.where(bcol == g, h2, -jnp.inf)
    mx = jnp.max(m, axis=0, keepdims=True)
    pooled[pl.ds(g, 1), :] = jnp.maximum(pooled[pl.ds(g, 1), :], mx)
    return carry

  lax.fori_loop(0, G, seg, 0)

  @pl.when(i == NBLK - 1)
  def _():
    out_ref[...] = jnp.dot(pooled[...], fcw_ref[...],
                           preferred_element_type=jnp.float32) + fcb_ref[...]


def _pool(a2, dis, b2r, batchc, fcWp, fcbp):
  return pl.pallas_call(
      _pool_body,
      grid=(NBLK,),
      in_specs=[
          pl.BlockSpec((NC, RB, F), lambda i: (0, i, 0)),
          pl.BlockSpec((RB, 1), lambda i: (i, 0)),
          pl.BlockSpec((1, O), lambda i: (0, 0)),
          pl.BlockSpec((RB, 1), lambda i: (i, 0)),
          pl.BlockSpec((O, 128), lambda i: (0, 0)),
          pl.BlockSpec((1, 128), lambda i: (0, 0)),
      ],
      out_specs=pl.BlockSpec((G, 128), lambda i: (0, 0)),
      out_shape=jax.ShapeDtypeStruct((G, 128), jnp.float32),
      scratch_shapes=[pltpu.VMEM((G, O), jnp.float32)],
  )(a2, dis, b2r, batchc, fcWp, fcbp)


@jax.jit
def kernel(x, edge_index, batch, W1, b1, W2, b2, fcW, fcb):
  pad_e = EPAD - E
  src = jnp.concatenate(
      [edge_index[0], jnp.full((pad_e,), N, jnp.int32)]).reshape(NT, AGG_CHUNKS,
                                                                 CHUNK)
  dst = jnp.concatenate(
      [edge_index[1], jnp.full((pad_e,), N, jnp.int32)]).reshape(NT, AGG_CHUNKS,
                                                                 CHUNK)
  dst_deg = dst.reshape(NC * NT, DEG_CHUNKS, CHUNK)
  xp = jnp.pad(x, ((0, NPAD - N), (0, 0)))
  batchc = jnp.pad(batch, (0, NPAD - N), constant_values=G)[:, None]
  b1r = b1[None, :]
  b2r = b2[None, :]
  fcWp = jnp.pad(fcW, ((0, 0), (0, 128 - C)))
  fcbp = jnp.pad(fcb, (0, 128 - C))[None, :]

  deg = _deg_kernel(dst_deg)
  deg3 = deg[:, :, None]
  u1, dis = _mm1(deg3, xp, W1)
  a1 = _agg_kernel(u1, src, dst)
  u2 = _mm2(a1, dis, b1r, W2)
  a2 = _agg_kernel(u2, src, dst)
  outp = _pool(a2, dis, b2r, batchc, fcWp, fcbp)
  return outp[:, :C]
